# baseline (speedup 1.0000x reference)
import jax
import jax.numpy as jnp
from jax.experimental import pallas as pl

_N = 50000
_E = 800000
_H = 64
_EBLK = 6400
_NBLK = 2000
_EG = _E // _EBLK
_NG = _N // _NBLK
_EPS = 1e-5


def _wr_stats(t, ps, pss):
    s = jnp.sum(t, axis=0, keepdims=True) / 8.0
    ss = jnp.sum(t * t, axis=0, keepdims=True) / 8.0
    ps[...] = jnp.broadcast_to(s, (8, t.shape[1]))
    pss[...] = jnp.broadcast_to(ss, (8, t.shape[1]))


def _proj1_k(xb, wT, b, hl, ps, pss):
    t = jnp.dot(xb[...], wT[...], preferred_element_type=jnp.float32) + b[...]
    hl[...] = t
    _wr_stats(t, ps, pss)


def _norm_relu_k(hl, scale, shift, out):
    out[...] = jnp.maximum(hl[...] * scale[...] + shift[...], 0.0)


def _edgeA_k(hr, hc, ea, waT, wbT, wcT, b, m1, ps, pss):
    t = jnp.dot(hr[...], waT[...], preferred_element_type=jnp.float32)
    t = t + jnp.dot(hc[...], wbT[...], preferred_element_type=jnp.float32)
    t = t + jnp.dot(ea[...], wcT[...], preferred_element_type=jnp.float32)
    t = t + b[...]
    m1[...] = t
    _wr_stats(t, ps, pss)


def _edgeB_k(m1, scale, shift, w2T, b2, out):
    y = jnp.maximum(m1[...] * scale[...] + shift[...], 0.0)
    out[...] = jnp.dot(y, w2T[...], preferred_element_type=jnp.float32) + b2[...]


def _node1_k(a, xr, pre, ps, pss):
    t = a[...] + xr[...]
    pre[...] = t
    _wr_stats(t, ps, pss)


def _node2_k(pre, xr, scale, shift, out):
    out[...] = jnp.maximum(pre[...] * scale[...] + shift[...], 0.0) + xr[...]


def _head_k(ge, c1wT, c1b, c2wT, c2b, c3wT, c3b, pwT, pb,
            logits, score, pe):
    g = ge[...]
    z = jnp.maximum(jnp.dot(g, c1wT[...], preferred_element_type=jnp.float32)
                    + c1b[...], 0.0)
    z = jnp.maximum(jnp.dot(z, c2wT[...], preferred_element_type=jnp.float32)
                    + c2b[...], 0.0)
    lg = jnp.dot(z, c3wT[...], preferred_element_type=jnp.float32) + c3b[...]
    logits[...] = lg
    score[...] = jax.nn.sigmoid(lg)
    pe[...] = jnp.dot(g, pwT[...], preferred_element_type=jnp.float32) + pb[...]


def _full(shape):
    return pl.BlockSpec(shape, lambda i: (0, 0))


def _row64():
    return pl.BlockSpec((8, _H), lambda i: (i, 0))


def _stats_to_scale_shift(ps, pss, n, g, be):
    mean = jnp.sum(ps, axis=0) / n
    var = jnp.sum(pss, axis=0) / n - mean * mean
    scale = g / jnp.sqrt(var + _EPS)
    shift = be - mean * scale
    return scale.reshape(1, -1), shift.reshape(1, -1)


def kernel(x, edge_index, edge_attr, batch, ip_w, ip_b, ip_g, ip_be,
           lw1, lb1, lg1, lbe1, lw2, lb2, ng, nb,
           c1w, c1b, c2w, c2b, c3w, c3b, pw, pb):
    f32 = jnp.float32
    row = edge_index[0]
    col = edge_index[1]

    hl, ps, pss = pl.pallas_call(
        _proj1_k,
        grid=(_NG,),
        in_specs=[pl.BlockSpec((_NBLK, x.shape[1]), lambda i: (i, 0)),
                  _full(ip_w.T.shape), _full((1, _H))],
        out_specs=(pl.BlockSpec((_NBLK, _H), lambda i: (i, 0)),
                   _row64(), _row64()),
        out_shape=(jax.ShapeDtypeStruct((_N, _H), f32),
                   jax.ShapeDtypeStruct((_NG * 8, _H), f32),
                   jax.ShapeDtypeStruct((_NG * 8, _H), f32)),
    )(x, ip_w.T, ip_b.reshape(1, -1))
    scale, shift = _stats_to_scale_shift(ps, pss, _N, ip_g, ip_be)
    h = pl.pallas_call(
        _norm_relu_k,
        grid=(_NG,),
        in_specs=[pl.BlockSpec((_NBLK, _H), lambda i: (i, 0)),
                  _full((1, _H)), _full((1, _H))],
        out_specs=pl.BlockSpec((_NBLK, _H), lambda i: (i, 0)),
        out_shape=jax.ShapeDtypeStruct((_N, _H), f32),
    )(hl, scale, shift)

    cnt = jax.ops.segment_sum(jnp.ones((_E,), f32), row, num_segments=_N)
    inv_cnt1 = (1.0 / (cnt + 1.0))[:, None]

    for i in range(lw1.shape[0]):
        x_res = h
        hr = jnp.take(h, row, axis=0)
        hc = jnp.take(h, col, axis=0)
        w1 = lw1[i]
        waT = w1[:, :_H].T
        wbT = w1[:, _H:2 * _H].T
        wcT = w1[:, 2 * _H:].T
        m1, ps, pss = pl.pallas_call(
            _edgeA_k,
            grid=(_EG,),
            in_specs=[pl.BlockSpec((_EBLK, _H), lambda j: (j, 0)),
                      pl.BlockSpec((_EBLK, _H), lambda j: (j, 0)),
                      pl.BlockSpec((_EBLK, edge_attr.shape[1]), lambda j: (j, 0)),
                      _full(waT.shape), _full(wbT.shape), _full(wcT.shape),
                      _full((1, _H))],
            out_specs=(pl.BlockSpec((_EBLK, _H), lambda j: (j, 0)),
                       _row64(), _row64()),
            out_shape=(jax.ShapeDtypeStruct((_E, _H), f32),
                       jax.ShapeDtypeStruct((_EG * 8, _H), f32),
                       jax.ShapeDtypeStruct((_EG * 8, _H), f32)),
        )(hr, hc, edge_attr, waT, wbT, wcT, lb1[i].reshape(1, -1))
        scale, shift = _stats_to_scale_shift(ps, pss, _E, lg1[i], lbe1[i])
        m = pl.pallas_call(
            _edgeB_k,
            grid=(_EG,),
            in_specs=[pl.BlockSpec((_EBLK, _H), lambda j: (j, 0)),
                      _full((1, _H)), _full((1, _H)),
                      _full((_H, _H)), _full((1, _H))],
            out_specs=pl.BlockSpec((_EBLK, _H), lambda j: (j, 0)),
            out_shape=jax.ShapeDtypeStruct((_E, _H), f32),
        )(m1, scale, shift, lw2[i].T, lb2[i].reshape(1, -1))
        sums = jax.ops.segment_sum(m, row, num_segments=_N)
        agg = sums * inv_cnt1
        pre, ps, pss = pl.pallas_call(
            _node1_k,
            grid=(_NG,),
            in_specs=[pl.BlockSpec((_NBLK, _H), lambda j: (j, 0)),
                      pl.BlockSpec((_NBLK, _H), lambda j: (j, 0))],
            out_specs=(pl.BlockSpec((_NBLK, _H), lambda j: (j, 0)),
                       _row64(), _row64()),
            out_shape=(jax.ShapeDtypeStruct((_N, _H), f32),
                       jax.ShapeDtypeStruct((_NG * 8, _H), f32),
                       jax.ShapeDtypeStruct((_NG * 8, _H), f32)),
        )(agg, x_res)
        scale, shift = _stats_to_scale_shift(ps, pss, _N, ng[i], nb[i])
        h = pl.pallas_call(
            _node2_k,
            grid=(_NG,),
            in_specs=[pl.BlockSpec((_NBLK, _H), lambda j: (j, 0)),
                      pl.BlockSpec((_NBLK, _H), lambda j: (j, 0)),
                      _full((1, _H)), _full((1, _H))],
            out_specs=pl.BlockSpec((_NBLK, _H), lambda j: (j, 0)),
            out_shape=jax.ShapeDtypeStruct((_N, _H), f32),
        )(pre, x_res, scale, shift)

    node_emb = h

    nbatch = 64
    cntb = jax.ops.segment_sum(jnp.ones((_N,), f32), batch, num_segments=nbatch)
    mean_p = jax.ops.segment_sum(h, batch, num_segments=nbatch) \
        / jnp.maximum(cntb, 1.0)[:, None]
    max_p = jax.ops.segment_max(h, batch, num_segments=nbatch)
    ge = jnp.concatenate([mean_p, max_p], axis=-1)

    logits, score, pe = pl.pallas_call(
        _head_k,
        out_shape=(jax.ShapeDtypeStruct((nbatch, 1), f32),
                   jax.ShapeDtypeStruct((nbatch, 1), f32),
                   jax.ShapeDtypeStruct((nbatch, pw.shape[0]), f32)),
    )(ge, c1w.T, c1b.reshape(1, -1), c2w.T, c2b.reshape(1, -1),
      c3w.T, c3b.reshape(1, -1), pw.T, pb.reshape(1, -1))
    return (logits, score, pe, node_emb)


# speedup vs baseline: 1.0197x; 1.0197x over previous
import jax
import jax.numpy as jnp
from jax.experimental import pallas as pl

_N = 50000
_E = 800000
_H = 64
_EBLK = 16000
_NBLK = 10000
_EG = _E // _EBLK
_NG = _N // _NBLK
_EPS = 1e-5


def _wr_stats(t, ps, pss):
    s = jnp.sum(t, axis=0, keepdims=True) / 8.0
    ss = jnp.sum(t * t, axis=0, keepdims=True) / 8.0
    ps[...] = jnp.broadcast_to(s, (8, t.shape[1]))
    pss[...] = jnp.broadcast_to(ss, (8, t.shape[1]))


def _proj1_k(xb, wT, b, hl, ps, pss):
    t = jnp.dot(xb[...], wT[...], preferred_element_type=jnp.float32) + b[...]
    hl[...] = t
    _wr_stats(t, ps, pss)


def _norm_relu_k(hl, scale, shift, out):
    out[...] = jnp.maximum(hl[...] * scale[...] + shift[...], 0.0)


def _edgeA_k(hr, hc, ea, waT, wbT, wcT, b, m1, ps, pss):
    t = jnp.dot(hr[...], waT[...], preferred_element_type=jnp.float32)
    t = t + jnp.dot(hc[...], wbT[...], preferred_element_type=jnp.float32)
    t = t + jnp.dot(ea[...], wcT[...], preferred_element_type=jnp.float32)
    t = t + b[...]
    m1[...] = t.astype(jnp.bfloat16)
    _wr_stats(t, ps, pss)


def _edgeB_k(m1, scale, shift, w2T, b2, out):
    y = jnp.maximum(m1[...].astype(jnp.float32) * scale[...] + shift[...], 0.0)
    t = jnp.dot(y.astype(jnp.bfloat16), w2T[...],
                preferred_element_type=jnp.float32) + b2[...]
    out[...] = t.astype(jnp.bfloat16)


def _node1_k(a, xr, pre, ps, pss):
    t = a[...] + xr[...]
    pre[...] = t
    _wr_stats(t, ps, pss)


def _node2_k(pre, xr, scale, shift, out):
    out[...] = jnp.maximum(pre[...] * scale[...] + shift[...], 0.0) + xr[...]


def _head_k(ge, c1wT, c1b, c2wT, c2b, c3wT, c3b, pwT, pb,
            logits, score, pe):
    g = ge[...]
    z = jnp.maximum(jnp.dot(g, c1wT[...], preferred_element_type=jnp.float32)
                    + c1b[...], 0.0)
    z = jnp.maximum(jnp.dot(z, c2wT[...], preferred_element_type=jnp.float32)
                    + c2b[...], 0.0)
    lg = jnp.dot(z, c3wT[...], preferred_element_type=jnp.float32) + c3b[...]
    logits[...] = lg
    score[...] = jax.nn.sigmoid(lg)
    pe[...] = jnp.dot(g, pwT[...], preferred_element_type=jnp.float32) + pb[...]


def _full(shape):
    return pl.BlockSpec(shape, lambda i: (0, 0))


def _row64():
    return pl.BlockSpec((8, _H), lambda i: (i, 0))


def _stats_to_scale_shift(ps, pss, n, g, be):
    mean = jnp.sum(ps, axis=0) / n
    var = jnp.sum(pss, axis=0) / n - mean * mean
    scale = g / jnp.sqrt(var + _EPS)
    shift = be - mean * scale
    return scale.reshape(1, -1), shift.reshape(1, -1)


def kernel(x, edge_index, edge_attr, batch, ip_w, ip_b, ip_g, ip_be,
           lw1, lb1, lg1, lbe1, lw2, lb2, ng, nb,
           c1w, c1b, c2w, c2b, c3w, c3b, pw, pb):
    f32 = jnp.float32
    row = edge_index[0]
    col = edge_index[1]

    hl, ps, pss = pl.pallas_call(
        _proj1_k,
        grid=(_NG,),
        in_specs=[pl.BlockSpec((_NBLK, x.shape[1]), lambda i: (i, 0)),
                  _full(ip_w.T.shape), _full((1, _H))],
        out_specs=(pl.BlockSpec((_NBLK, _H), lambda i: (i, 0)),
                   _row64(), _row64()),
        out_shape=(jax.ShapeDtypeStruct((_N, _H), f32),
                   jax.ShapeDtypeStruct((_NG * 8, _H), f32),
                   jax.ShapeDtypeStruct((_NG * 8, _H), f32)),
    )(x, ip_w.T, ip_b.reshape(1, -1))
    scale, shift = _stats_to_scale_shift(ps, pss, _N, ip_g, ip_be)
    h = pl.pallas_call(
        _norm_relu_k,
        grid=(_NG,),
        in_specs=[pl.BlockSpec((_NBLK, _H), lambda i: (i, 0)),
                  _full((1, _H)), _full((1, _H))],
        out_specs=pl.BlockSpec((_NBLK, _H), lambda i: (i, 0)),
        out_shape=jax.ShapeDtypeStruct((_N, _H), f32),
    )(hl, scale, shift)

    cnt = jax.ops.segment_sum(jnp.ones((_E,), f32), row, num_segments=_N)
    inv_cnt1 = (1.0 / (cnt + 1.0))[:, None]

    bf16 = jnp.bfloat16
    ea_bf = edge_attr.astype(bf16)
    for i in range(lw1.shape[0]):
        x_res = h
        h_bf = h.astype(bf16)
        hr = jnp.take(h_bf, row, axis=0)
        hc = jnp.take(h_bf, col, axis=0)
        w1 = lw1[i]
        waT = w1[:, :_H].T.astype(bf16)
        wbT = w1[:, _H:2 * _H].T.astype(bf16)
        wcT = w1[:, 2 * _H:].T.astype(bf16)
        m1, ps, pss = pl.pallas_call(
            _edgeA_k,
            grid=(_EG,),
            in_specs=[pl.BlockSpec((_EBLK, _H), lambda j: (j, 0)),
                      pl.BlockSpec((_EBLK, _H), lambda j: (j, 0)),
                      pl.BlockSpec((_EBLK, ea_bf.shape[1]), lambda j: (j, 0)),
                      _full(waT.shape), _full(wbT.shape), _full(wcT.shape),
                      _full((1, _H))],
            out_specs=(pl.BlockSpec((_EBLK, _H), lambda j: (j, 0)),
                       _row64(), _row64()),
            out_shape=(jax.ShapeDtypeStruct((_E, _H), jnp.bfloat16),
                       jax.ShapeDtypeStruct((_EG * 8, _H), f32),
                       jax.ShapeDtypeStruct((_EG * 8, _H), f32)),
        )(hr, hc, ea_bf, waT, wbT, wcT, lb1[i].reshape(1, -1))
        scale, shift = _stats_to_scale_shift(ps, pss, _E, lg1[i], lbe1[i])
        m = pl.pallas_call(
            _edgeB_k,
            grid=(_EG,),
            in_specs=[pl.BlockSpec((_EBLK, _H), lambda j: (j, 0)),
                      _full((1, _H)), _full((1, _H)),
                      _full((_H, _H)), _full((1, _H))],
            out_specs=pl.BlockSpec((_EBLK, _H), lambda j: (j, 0)),
            out_shape=jax.ShapeDtypeStruct((_E, _H), jnp.bfloat16),
        )(m1, scale, shift, lw2[i].T.astype(bf16), lb2[i].reshape(1, -1))
        sums = jax.ops.segment_sum(m.astype(f32), row, num_segments=_N)
        agg = sums * inv_cnt1
        pre, ps, pss = pl.pallas_call(
            _node1_k,
            grid=(_NG,),
            in_specs=[pl.BlockSpec((_NBLK, _H), lambda j: (j, 0)),
                      pl.BlockSpec((_NBLK, _H), lambda j: (j, 0))],
            out_specs=(pl.BlockSpec((_NBLK, _H), lambda j: (j, 0)),
                       _row64(), _row64()),
            out_shape=(jax.ShapeDtypeStruct((_N, _H), f32),
                       jax.ShapeDtypeStruct((_NG * 8, _H), f32),
                       jax.ShapeDtypeStruct((_NG * 8, _H), f32)),
        )(agg, x_res)
        scale, shift = _stats_to_scale_shift(ps, pss, _N, ng[i], nb[i])
        h = pl.pallas_call(
            _node2_k,
            grid=(_NG,),
            in_specs=[pl.BlockSpec((_NBLK, _H), lambda j: (j, 0)),
                      pl.BlockSpec((_NBLK, _H), lambda j: (j, 0)),
                      _full((1, _H)), _full((1, _H))],
            out_specs=pl.BlockSpec((_NBLK, _H), lambda j: (j, 0)),
            out_shape=jax.ShapeDtypeStruct((_N, _H), f32),
        )(pre, x_res, scale, shift)

    node_emb = h

    nbatch = 64
    cntb = jax.ops.segment_sum(jnp.ones((_N,), f32), batch, num_segments=nbatch)
    mean_p = jax.ops.segment_sum(h, batch, num_segments=nbatch) \
        / jnp.maximum(cntb, 1.0)[:, None]
    max_p = jax.ops.segment_max(h, batch, num_segments=nbatch)
    ge = jnp.concatenate([mean_p, max_p], axis=-1)

    logits, score, pe = pl.pallas_call(
        _head_k,
        out_shape=(jax.ShapeDtypeStruct((nbatch, 1), f32),
                   jax.ShapeDtypeStruct((nbatch, 1), f32),
                   jax.ShapeDtypeStruct((nbatch, pw.shape[0]), f32)),
    )(ge, c1w.T, c1b.reshape(1, -1), c2w.T, c2b.reshape(1, -1),
      c3w.T, c3b.reshape(1, -1), pw.T, pb.reshape(1, -1))
    return (logits, score, pe, node_emb)


# speedup vs baseline: 1.0458x; 1.0256x over previous
import jax
import jax.numpy as jnp
from jax.experimental import pallas as pl

_N = 50000
_E = 800000
_H = 64
_EBLK = 16000
_NBLK = 10000
_EG = _E // _EBLK
_NG = _N // _NBLK
_EPS = 1e-5


def _wr_stats(t, ps, pss):
    s = jnp.sum(t, axis=0, keepdims=True) / 8.0
    ss = jnp.sum(t * t, axis=0, keepdims=True) / 8.0
    ps[...] = jnp.broadcast_to(s, (8, t.shape[1]))
    pss[...] = jnp.broadcast_to(ss, (8, t.shape[1]))


def _proj1_k(xb, wT, b, hl, ps, pss):
    t = jnp.dot(xb[...], wT[...], preferred_element_type=jnp.float32) + b[...]
    hl[...] = t
    _wr_stats(t, ps, pss)


def _norm_relu_k(hl, scale, shift, out):
    out[...] = jnp.maximum(hl[...] * scale[...] + shift[...], 0.0)


def _edgeA_k(hr, hc, ea, waT, wbT, wcT, b, m1, ps, pss):
    t = jnp.dot(hr[...], waT[...], preferred_element_type=jnp.float32)
    t = t + jnp.dot(hc[...], wbT[...], preferred_element_type=jnp.float32)
    t = t + jnp.dot(ea[...], wcT[...], preferred_element_type=jnp.float32)
    t = t + b[...]
    m1[...] = t.astype(jnp.bfloat16)
    _wr_stats(t, ps, pss)


def _edgeB_k(m1, scale, shift, w2T, b2, out):
    y = jnp.maximum(m1[...].astype(jnp.float32) * scale[...] + shift[...], 0.0)
    out[...] = jnp.dot(y.astype(jnp.bfloat16), w2T[...],
                       preferred_element_type=jnp.float32) + b2[...]


def _node1_k(a, xr, pre, ps, pss):
    t = a[...] + xr[...]
    pre[...] = t
    _wr_stats(t, ps, pss)


def _node2_k(pre, xr, scale, shift, out):
    out[...] = jnp.maximum(pre[...] * scale[...] + shift[...], 0.0) + xr[...]


def _head_k(ge, c1wT, c1b, c2wT, c2b, c3wT, c3b, pwT, pb,
            logits, score, pe):
    g = ge[...]
    z = jnp.maximum(jnp.dot(g, c1wT[...], preferred_element_type=jnp.float32)
                    + c1b[...], 0.0)
    z = jnp.maximum(jnp.dot(z, c2wT[...], preferred_element_type=jnp.float32)
                    + c2b[...], 0.0)
    lg = jnp.dot(z, c3wT[...], preferred_element_type=jnp.float32) + c3b[...]
    logits[...] = lg
    score[...] = jax.nn.sigmoid(lg)
    pe[...] = jnp.dot(g, pwT[...], preferred_element_type=jnp.float32) + pb[...]


def _full(shape):
    return pl.BlockSpec(shape, lambda i: (0, 0))


def _row64():
    return pl.BlockSpec((8, _H), lambda i: (i, 0))


def _stats_to_scale_shift(ps, pss, n, g, be):
    mean = jnp.sum(ps, axis=0) / n
    var = jnp.sum(pss, axis=0) / n - mean * mean
    scale = g / jnp.sqrt(var + _EPS)
    shift = be - mean * scale
    return scale.reshape(1, -1), shift.reshape(1, -1)


def kernel(x, edge_index, edge_attr, batch, ip_w, ip_b, ip_g, ip_be,
           lw1, lb1, lg1, lbe1, lw2, lb2, ng, nb,
           c1w, c1b, c2w, c2b, c3w, c3b, pw, pb):
    f32 = jnp.float32
    row = edge_index[0]
    col = edge_index[1]

    hl, ps, pss = pl.pallas_call(
        _proj1_k,
        grid=(_NG,),
        in_specs=[pl.BlockSpec((_NBLK, x.shape[1]), lambda i: (i, 0)),
                  _full(ip_w.T.shape), _full((1, _H))],
        out_specs=(pl.BlockSpec((_NBLK, _H), lambda i: (i, 0)),
                   _row64(), _row64()),
        out_shape=(jax.ShapeDtypeStruct((_N, _H), f32),
                   jax.ShapeDtypeStruct((_NG * 8, _H), f32),
                   jax.ShapeDtypeStruct((_NG * 8, _H), f32)),
    )(x, ip_w.T, ip_b.reshape(1, -1))
    scale, shift = _stats_to_scale_shift(ps, pss, _N, ip_g, ip_be)
    h = pl.pallas_call(
        _norm_relu_k,
        grid=(_NG,),
        in_specs=[pl.BlockSpec((_NBLK, _H), lambda i: (i, 0)),
                  _full((1, _H)), _full((1, _H))],
        out_specs=pl.BlockSpec((_NBLK, _H), lambda i: (i, 0)),
        out_shape=jax.ShapeDtypeStruct((_N, _H), f32),
    )(hl, scale, shift)

    cnt = jax.ops.segment_sum(jnp.ones((_E,), f32), row, num_segments=_N)
    inv_cnt1 = (1.0 / (cnt + 1.0))[:, None]

    bf16 = jnp.bfloat16
    ea_bf = edge_attr.astype(bf16)
    for i in range(lw1.shape[0]):
        x_res = h
        h_bf = h.astype(bf16)
        hr = jnp.take(h_bf, row, axis=0)
        hc = jnp.take(h_bf, col, axis=0)
        w1 = lw1[i]
        waT = w1[:, :_H].T.astype(bf16)
        wbT = w1[:, _H:2 * _H].T.astype(bf16)
        wcT = w1[:, 2 * _H:].T.astype(bf16)
        m1, ps, pss = pl.pallas_call(
            _edgeA_k,
            grid=(_EG,),
            in_specs=[pl.BlockSpec((_EBLK, _H), lambda j: (j, 0)),
                      pl.BlockSpec((_EBLK, _H), lambda j: (j, 0)),
                      pl.BlockSpec((_EBLK, ea_bf.shape[1]), lambda j: (j, 0)),
                      _full(waT.shape), _full(wbT.shape), _full(wcT.shape),
                      _full((1, _H))],
            out_specs=(pl.BlockSpec((_EBLK, _H), lambda j: (j, 0)),
                       _row64(), _row64()),
            out_shape=(jax.ShapeDtypeStruct((_E, _H), jnp.bfloat16),
                       jax.ShapeDtypeStruct((_EG * 8, _H), f32),
                       jax.ShapeDtypeStruct((_EG * 8, _H), f32)),
        )(hr, hc, ea_bf, waT, wbT, wcT, lb1[i].reshape(1, -1))
        scale, shift = _stats_to_scale_shift(ps, pss, _E, lg1[i], lbe1[i])
        m = pl.pallas_call(
            _edgeB_k,
            grid=(_EG,),
            in_specs=[pl.BlockSpec((_EBLK, _H), lambda j: (j, 0)),
                      _full((1, _H)), _full((1, _H)),
                      _full((_H, _H)), _full((1, _H))],
            out_specs=pl.BlockSpec((_EBLK, _H), lambda j: (j, 0)),
            out_shape=jax.ShapeDtypeStruct((_E, _H), f32),
        )(m1, scale, shift, lw2[i].T.astype(bf16), lb2[i].reshape(1, -1))
        sums = jax.ops.segment_sum(m, row, num_segments=_N)
        agg = sums * inv_cnt1
        pre, ps, pss = pl.pallas_call(
            _node1_k,
            grid=(_NG,),
            in_specs=[pl.BlockSpec((_NBLK, _H), lambda j: (j, 0)),
                      pl.BlockSpec((_NBLK, _H), lambda j: (j, 0))],
            out_specs=(pl.BlockSpec((_NBLK, _H), lambda j: (j, 0)),
                       _row64(), _row64()),
            out_shape=(jax.ShapeDtypeStruct((_N, _H), f32),
                       jax.ShapeDtypeStruct((_NG * 8, _H), f32),
                       jax.ShapeDtypeStruct((_NG * 8, _H), f32)),
        )(agg, x_res)
        scale, shift = _stats_to_scale_shift(ps, pss, _N, ng[i], nb[i])
        h = pl.pallas_call(
            _node2_k,
            grid=(_NG,),
            in_specs=[pl.BlockSpec((_NBLK, _H), lambda j: (j, 0)),
                      pl.BlockSpec((_NBLK, _H), lambda j: (j, 0)),
                      _full((1, _H)), _full((1, _H))],
            out_specs=pl.BlockSpec((_NBLK, _H), lambda j: (j, 0)),
            out_shape=jax.ShapeDtypeStruct((_N, _H), f32),
        )(pre, x_res, scale, shift)

    node_emb = h

    nbatch = 64
    cntb = jax.ops.segment_sum(jnp.ones((_N,), f32), batch, num_segments=nbatch)
    mean_p = jax.ops.segment_sum(h, batch, num_segments=nbatch) \
        / jnp.maximum(cntb, 1.0)[:, None]
    max_p = jax.ops.segment_max(h, batch, num_segments=nbatch)
    ge = jnp.concatenate([mean_p, max_p], axis=-1)

    logits, score, pe = pl.pallas_call(
        _head_k,
        out_shape=(jax.ShapeDtypeStruct((nbatch, 1), f32),
                   jax.ShapeDtypeStruct((nbatch, 1), f32),
                   jax.ShapeDtypeStruct((nbatch, pw.shape[0]), f32)),
    )(ge, c1w.T, c1b.reshape(1, -1), c2w.T, c2b.reshape(1, -1),
      c3w.T, c3b.reshape(1, -1), pw.T, pb.reshape(1, -1))
    return (logits, score, pe, node_emb)
